# baseline (speedup 1.0000x reference)
import functools

import jax
import jax.numpy as jnp
from jax import lax
from jax.experimental import pallas as pl
from jax.experimental.pallas import tpu as pltpu

_NSTREAM = 8
_UNROLL = 8


def _nn_body(pred_ref, predI_ref, tgt_ref, outI_ref, outidx_ref):
    T = tgt_ref[0]
    S, LN = T.shape
    n_pred = pred_ref.shape[2]
    per_iter = _NSTREAM * _UNROLL

    def step(k, carry):
        rmin, ridx, rI = carry
        rmin, ridx, rI = list(rmin), list(ridx), list(rI)
        for u in range(_UNROLL):
            for r in range(_NSTREAM):
                i = k * per_iter + u * _NSTREAM + r
                p = pred_ref[0, 0, i]
                iv = predI_ref[0, 0, i]
                d = jnp.abs(T - p)
                upd = d < rmin[r]
                rmin[r] = jnp.minimum(d, rmin[r])
                ridx[r] = jnp.where(upd, i, ridx[r])
                rI[r] = jnp.where(upd, iv, rI[r])
        return tuple(rmin), tuple(ridx), tuple(rI)

    init = (
        tuple(jnp.full((S, LN), jnp.inf, jnp.float32) for _ in range(_NSTREAM)),
        tuple(jnp.zeros((S, LN), jnp.int32) for _ in range(_NSTREAM)),
        tuple(jnp.zeros((S, LN), jnp.float32) for _ in range(_NSTREAM)),
    )
    rmin, ridx, rI = lax.fori_loop(0, n_pred // per_iter, step, init)

    bd, bi, bI = rmin[0], ridx[0], rI[0]
    for r in range(1, _NSTREAM):
        better = (rmin[r] < bd) | ((rmin[r] == bd) & (ridx[r] < bi))
        bd = jnp.where(better, rmin[r], bd)
        bi = jnp.where(better, ridx[r], bi)
        bI = jnp.where(better, rI[r], bI)
    outI_ref[0] = bI
    outidx_ref[0] = bi


@functools.partial(jax.jit, static_argnames=("interpret",))
def _nn_match(pred_mz, pred_I, tgt_mz, interpret=False):
    B, Np = pred_mz.shape
    _, Nt = tgt_mz.shape
    S = 8
    LN = Nt // S
    tgt3 = tgt_mz.reshape(B, S, LN)
    pred3 = pred_mz.reshape(B, 1, Np)
    predI3 = pred_I.reshape(B, 1, Np)
    matched_I3, matched_idx3 = pl.pallas_call(
        _nn_body,
        grid=(B,),
        in_specs=[
            pl.BlockSpec((1, 1, Np), lambda b: (b, 0, 0), memory_space=pltpu.SMEM),
            pl.BlockSpec((1, 1, Np), lambda b: (b, 0, 0), memory_space=pltpu.SMEM),
            pl.BlockSpec((1, S, LN), lambda b: (b, 0, 0)),
        ],
        out_specs=(
            pl.BlockSpec((1, S, LN), lambda b: (b, 0, 0)),
            pl.BlockSpec((1, S, LN), lambda b: (b, 0, 0)),
        ),
        out_shape=(
            jax.ShapeDtypeStruct((B, S, LN), jnp.float32),
            jax.ShapeDtypeStruct((B, S, LN), jnp.int32),
        ),
        interpret=interpret,
    )(pred3, predI3, tgt3)
    return matched_I3.reshape(B, Nt), matched_idx3.reshape(B, Nt)


def kernel(pred_mz, pred_I, tgt_mz):
    return _nn_match(pred_mz, pred_I, tgt_mz)


# speedup vs baseline: 1.0535x; 1.0535x over previous
import functools

import jax
import jax.numpy as jnp
from jax import lax
from jax.experimental import pallas as pl
from jax.experimental.pallas import tpu as pltpu

_NSTREAM = 4
_UNROLL = 32


def _nn_body(pred_ref, predI_ref, tgt_ref, outI_ref, outidx_ref):
    T = tgt_ref[0]
    S, LN = T.shape
    n_pred = pred_ref.shape[2]
    per_iter = _NSTREAM * _UNROLL

    def step(k, carry):
        rmin, ridx, rI = carry
        rmin, ridx, rI = list(rmin), list(ridx), list(rI)
        for u in range(_UNROLL):
            for r in range(_NSTREAM):
                i = k * per_iter + u * _NSTREAM + r
                p = pred_ref[0, 0, i]
                iv = predI_ref[0, 0, i]
                d = jnp.abs(T - p)
                upd = d < rmin[r]
                rmin[r] = jnp.minimum(d, rmin[r])
                ridx[r] = jnp.where(upd, i, ridx[r])
                rI[r] = jnp.where(upd, iv, rI[r])
        return tuple(rmin), tuple(ridx), tuple(rI)

    init = (
        tuple(jnp.full((S, LN), jnp.inf, jnp.float32) for _ in range(_NSTREAM)),
        tuple(jnp.zeros((S, LN), jnp.int32) for _ in range(_NSTREAM)),
        tuple(jnp.zeros((S, LN), jnp.float32) for _ in range(_NSTREAM)),
    )
    rmin, ridx, rI = lax.fori_loop(0, n_pred // per_iter, step, init)

    bd, bi, bI = rmin[0], ridx[0], rI[0]
    for r in range(1, _NSTREAM):
        better = (rmin[r] < bd) | ((rmin[r] == bd) & (ridx[r] < bi))
        bd = jnp.where(better, rmin[r], bd)
        bi = jnp.where(better, ridx[r], bi)
        bI = jnp.where(better, rI[r], bI)
    outI_ref[0] = bI
    outidx_ref[0] = bi


@functools.partial(jax.jit, static_argnames=("interpret",))
def _nn_match(pred_mz, pred_I, tgt_mz, interpret=False):
    B, Np = pred_mz.shape
    _, Nt = tgt_mz.shape
    S = 8
    LN = Nt // S
    tgt3 = tgt_mz.reshape(B, S, LN)
    pred3 = pred_mz.reshape(B, 1, Np)
    predI3 = pred_I.reshape(B, 1, Np)
    matched_I3, matched_idx3 = pl.pallas_call(
        _nn_body,
        grid=(B,),
        in_specs=[
            pl.BlockSpec((1, 1, Np), lambda b: (b, 0, 0), memory_space=pltpu.SMEM),
            pl.BlockSpec((1, 1, Np), lambda b: (b, 0, 0), memory_space=pltpu.SMEM),
            pl.BlockSpec((1, S, LN), lambda b: (b, 0, 0)),
        ],
        out_specs=(
            pl.BlockSpec((1, S, LN), lambda b: (b, 0, 0)),
            pl.BlockSpec((1, S, LN), lambda b: (b, 0, 0)),
        ),
        out_shape=(
            jax.ShapeDtypeStruct((B, S, LN), jnp.float32),
            jax.ShapeDtypeStruct((B, S, LN), jnp.int32),
        ),
        interpret=interpret,
    )(pred3, predI3, tgt3)
    return matched_I3.reshape(B, Nt), matched_idx3.reshape(B, Nt)


def kernel(pred_mz, pred_I, tgt_mz):
    return _nn_match(pred_mz, pred_I, tgt_mz)
